# baseline (device time: 133761 ns/iter reference)
import jax
import jax.numpy as jnp
from jax import lax
from jax.experimental import pallas as pl
from jax.experimental.pallas import tpu as pltpu

N_DEV = 8
M_PER = 512
N_HOP = 3
NSUB = 4
SUB = M_PER // NSUB


def kernel(x, w_mat, scale_x, scale_w):
    m_per, k = x.shape
    _, n_per = w_mat.shape
    scale = (scale_x[0] * scale_w[0]).reshape(1, 1)

    def body(x_ref, w_ref, scale_ref, out_ref,
             allx_ref, fwd_send, fwd_recv, bwd_send, bwd_recv,
             z_send, z_recv):
        my = lax.axis_index("i")
        right = (my + 1) % N_DEV
        left = (my + N_DEV - 1) % N_DEV
        anti = (my + 4) % N_DEV

        barrier = pltpu.get_barrier_semaphore()
        for nbr in (left, right, anti):
            pl.semaphore_signal(barrier, inc=1, device_id=(nbr,),
                                device_id_type=pl.DeviceIdType.MESH)
        pl.semaphore_wait(barrier, 3)

        def gemm(chunk, row0):
            acc = lax.dot_general(
                chunk.astype(jnp.bfloat16), w_ref[...].astype(jnp.bfloat16),
                dimension_numbers=(((1,), (0,)), ((), ())),
                preferred_element_type=jnp.float32,
            )
            out_ref[pl.ds(row0, chunk.shape[0]), :] = acc * scale_ref[0, 0]

        def rdma(origin, s, send_sems, recv_sems, h, dst):
            return pltpu.make_async_remote_copy(
                src_ref=allx_ref.at[origin, s], dst_ref=allx_ref.at[origin, s],
                send_sem=send_sems.at[h, s], recv_sem=recv_sems.at[h, s],
                device_id=(dst,), device_id_type=pl.DeviceIdType.MESH,
            )

        allx_ref[my] = x_ref[...].reshape(NSUB, SUB, k)

        for s in range(NSUB):
            rdma(my, s, z_send, z_recv, 0, anti).start()
            rdma(my, s, fwd_send, fwd_recv, 0, right).start()
            rdma(my, s, bwd_send, bwd_recv, 0, left).start()

        gemm(x_ref[...], my * M_PER)

        for h in range(N_HOP):
            of_r = (my + N_DEV - h - 1) % N_DEV
            ob_r = (my + h + 1) % N_DEV

            for s in range(NSUB):
                rdma(of_r, s, fwd_send, fwd_recv, h, right).wait_recv()
                if h < N_HOP - 1:
                    rdma(of_r, s, fwd_send, fwd_recv, h + 1, right).start()
                rdma(ob_r, s, bwd_send, bwd_recv, h, left).wait_recv()
                if h < N_HOP - 1:
                    rdma(ob_r, s, bwd_send, bwd_recv, h + 1, left).start()

            gemm(allx_ref[of_r].reshape(M_PER, k), of_r * M_PER)
            gemm(allx_ref[ob_r].reshape(M_PER, k), ob_r * M_PER)

            if h == 0:
                for s in range(NSUB):
                    rdma(anti, s, z_send, z_recv, 0, anti).wait_recv()
                gemm(allx_ref[anti].reshape(M_PER, k), anti * M_PER)

        for h in range(N_HOP):
            for s in range(NSUB):
                rdma((my + N_DEV - h) % N_DEV, s, fwd_send, fwd_recv, h,
                     right).wait_send()
                rdma((my + h) % N_DEV, s, bwd_send, bwd_recv, h,
                     left).wait_send()
        for s in range(NSUB):
            rdma(my, s, z_send, z_recv, 0, anti).wait_send()

    return pl.pallas_call(
        body,
        out_shape=jax.ShapeDtypeStruct((N_DEV * m_per, n_per), jnp.float32),
        in_specs=[
            pl.BlockSpec(memory_space=pltpu.VMEM),
            pl.BlockSpec(memory_space=pltpu.VMEM),
            pl.BlockSpec(memory_space=pltpu.SMEM),
        ],
        out_specs=pl.BlockSpec(memory_space=pltpu.VMEM),
        scratch_shapes=[
            pltpu.VMEM((N_DEV, NSUB, SUB, k), jnp.int8),
            pltpu.SemaphoreType.DMA((N_HOP, NSUB)),
            pltpu.SemaphoreType.DMA((N_HOP, NSUB)),
            pltpu.SemaphoreType.DMA((N_HOP, NSUB)),
            pltpu.SemaphoreType.DMA((N_HOP, NSUB)),
            pltpu.SemaphoreType.DMA((1, NSUB)),
            pltpu.SemaphoreType.DMA((1, NSUB)),
        ],
        compiler_params=pltpu.CompilerParams(
            collective_id=0, vmem_limit_bytes=100 * 1024 * 1024,
        ),
    )(x, w_mat, scale)


# device time: 105278 ns/iter; 1.2706x vs baseline; 1.2706x over previous
import jax
import jax.numpy as jnp
from jax import lax
from jax.experimental import pallas as pl
from jax.experimental.pallas import tpu as pltpu

N_DEV = 8
M_PER = 512
NSUB = 2
SUB = M_PER // NSUB

FA, BA, Z, FB, FD, BC, BD = range(7)


def kernel(x, w_mat, scale_x, scale_w):
    m_per, k = x.shape
    _, n_per = w_mat.shape
    scale = (scale_x[0] * scale_w[0]).reshape(1, 1)

    def body(x_ref, w_ref, scale_ref, out_ref, allx_ref, send_sems, recv_sems):
        my = lax.axis_index("i")
        j4 = lax.rem(my, 4)
        base = my - j4
        right = base + lax.rem(j4 + 1, 4)
        left = base + lax.rem(j4 + 3, 4)
        anti = lax.rem(my + 4, N_DEV)
        ll = base + lax.rem(j4 + 2, 4)
        l_anti = lax.rem(left + 4, N_DEV)
        r_anti = lax.rem(right + 4, N_DEV)
        rr_anti = lax.rem(ll + 4, N_DEV)

        barrier = pltpu.get_barrier_semaphore()
        for nbr in (left, right, anti):
            pl.semaphore_signal(barrier, inc=1, device_id=(nbr,),
                                device_id_type=pl.DeviceIdType.MESH)
        pl.semaphore_wait(barrier, 3)

        def gemm(origin):
            acc = lax.dot_general(
                allx_ref[origin].reshape(M_PER, k).astype(jnp.bfloat16),
                w_ref[...].astype(jnp.bfloat16),
                dimension_numbers=(((1,), (0,)), ((), ())),
                preferred_element_type=jnp.float32,
            )
            out_ref[pl.ds(origin * M_PER, M_PER), :] = acc * scale_ref[0, 0]

        def rdma(origin, flow, s, dst):
            return pltpu.make_async_remote_copy(
                src_ref=allx_ref.at[origin, s], dst_ref=allx_ref.at[origin, s],
                send_sem=send_sems.at[flow, s], recv_sem=recv_sems.at[flow, s],
                device_id=(dst,), device_id_type=pl.DeviceIdType.MESH,
            )

        allx_ref[my] = x_ref[...].reshape(NSUB, SUB, k)

        for s in range(NSUB):
            rdma(my, FA, s, right).start()
            rdma(my, BA, s, left).start()
            rdma(my, Z, s, anti).start()

        acc = lax.dot_general(
            x_ref[...].astype(jnp.bfloat16), w_ref[...].astype(jnp.bfloat16),
            dimension_numbers=(((1,), (0,)), ((), ())),
            preferred_element_type=jnp.float32,
        )
        out_ref[pl.ds(my * M_PER, M_PER), :] = acc * scale_ref[0, 0]

        for s in range(NSUB):
            rdma(left, FA, s, right).wait_recv()
            rdma(left, FB, s, right).start()
        gemm(left)

        for s in range(NSUB):
            rdma(anti, Z, s, anti).wait_recv()
            rdma(anti, FD, s, right).start()
            rdma(anti, BC, s, left).start()
        gemm(anti)

        for s in range(NSUB):
            rdma(right, BA, s, left).wait_recv()
        gemm(right)

        for s in range(NSUB):
            rdma(r_anti, BC, s, left).wait_recv()
            rdma(r_anti, BD, s, left).start()
        gemm(r_anti)

        for s in range(NSUB):
            rdma(ll, FB, s, right).wait_recv()
        gemm(ll)

        for s in range(NSUB):
            rdma(l_anti, FD, s, right).wait_recv()
        gemm(l_anti)

        for s in range(NSUB):
            rdma(rr_anti, BD, s, left).wait_recv()
        gemm(rr_anti)

        for s in range(NSUB):
            rdma(my, FA, s, right).wait_send()
            rdma(my, BA, s, left).wait_send()
            rdma(my, Z, s, anti).wait_send()
            rdma(left, FB, s, right).wait_send()
            rdma(anti, FD, s, right).wait_send()
            rdma(anti, BC, s, left).wait_send()
            rdma(r_anti, BD, s, left).wait_send()

    return pl.pallas_call(
        body,
        out_shape=jax.ShapeDtypeStruct((N_DEV * m_per, n_per), jnp.float32),
        in_specs=[
            pl.BlockSpec(memory_space=pltpu.VMEM),
            pl.BlockSpec(memory_space=pltpu.VMEM),
            pl.BlockSpec(memory_space=pltpu.SMEM),
        ],
        out_specs=pl.BlockSpec(memory_space=pltpu.VMEM),
        scratch_shapes=[
            pltpu.VMEM((N_DEV, NSUB, SUB, k), jnp.int8),
            pltpu.SemaphoreType.DMA((7, NSUB)),
            pltpu.SemaphoreType.DMA((7, NSUB)),
        ],
        compiler_params=pltpu.CompilerParams(
            collective_id=0, vmem_limit_bytes=100 * 1024 * 1024,
        ),
    )(x, w_mat, scale)
